# baseline (speedup 1.0000x reference)
import functools

import jax
import jax.numpy as jnp
from jax import lax
from jax.experimental import pallas as pl
from jax.experimental.pallas import tpu as pltpu
from jax.experimental.pallas import tpu_sc as plsc

_NODE = 64
_HEADS = 4
_DH = 16
_NBLK = 400
_EBLK = 3200


def _softplus(x):
    return jnp.maximum(x, 0.0) + jnp.log1p(jnp.exp(-jnp.abs(x)))


def _sigmoid(x):
    return 1.0 / (1.0 + jnp.exp(-x))


def _mm_kernel(x_ref, w_ref, b_ref, o_ref):
    o_ref[...] = (
        jnp.dot(x_ref[...], w_ref[...], preferred_element_type=jnp.float32)
        + b_ref[...]
    )


def _mm(x, w, b, blk=_NBLK):
    n, k = x.shape
    m = w.shape[1]
    return pl.pallas_call(
        _mm_kernel,
        grid=(n // blk,),
        in_specs=[
            pl.BlockSpec((blk, k), lambda i: (i, 0)),
            pl.BlockSpec((k, m), lambda i: (0, 0)),
            pl.BlockSpec((1, m), lambda i: (0, 0)),
        ],
        out_specs=pl.BlockSpec((blk, m), lambda i: (i, 0)),
        out_shape=jax.ShapeDtypeStruct((n, m), jnp.float32),
    )(x, w, b)


def _mm2_kernel(a_ref, b_ref, wa_ref, wb_ref, bias_ref, o_ref):
    o_ref[...] = (
        jnp.dot(a_ref[...], wa_ref[...], preferred_element_type=jnp.float32)
        + jnp.dot(b_ref[...], wb_ref[...], preferred_element_type=jnp.float32)
        + bias_ref[...]
    )


def _mm2(a, b, wa, wb, bias, blk=_NBLK):
    n, ka = a.shape
    kb = b.shape[1]
    m = wa.shape[1]
    return pl.pallas_call(
        _mm2_kernel,
        grid=(n // blk,),
        in_specs=[
            pl.BlockSpec((blk, ka), lambda i: (i, 0)),
            pl.BlockSpec((blk, kb), lambda i: (i, 0)),
            pl.BlockSpec((ka, m), lambda i: (0, 0)),
            pl.BlockSpec((kb, m), lambda i: (0, 0)),
            pl.BlockSpec((1, m), lambda i: (0, 0)),
        ],
        out_specs=pl.BlockSpec((blk, m), lambda i: (i, 0)),
        out_shape=jax.ShapeDtypeStruct((n, m), jnp.float32),
    )(a, b, wa, wb, bias)


def _combiner_kernel(cnt_ref, s_ref, s2_ref, mx_ref, mn_ref,
                     lwT_ref, lb_ref, ipwT_ref, ipb_ref, opwT_ref, opb_ref,
                     o_ref):
    cnt = cnt_ref[...]
    cnt_c = jnp.maximum(cnt, 1.0)
    mean = s_ref[...] / cnt_c
    var = s2_ref[...] / cnt_c - mean * mean
    std = jnp.sqrt(jnp.clip(var, 1e-5, None))
    pos = cnt > 0.0
    mx = jnp.where(pos, mx_ref[...], 0.0)
    mn = jnp.where(pos, mn_ref[...], 0.0)
    aggs = (mean, std, mx, mn)

    ipwT = ipwT_ref[...]
    ipb = ipb_ref[...]
    qs, ks, vs = [], [], []
    for k in range(4):
        p = (jnp.dot(aggs[k], lwT_ref[k], preferred_element_type=jnp.float32)
             + lb_ref[k])
        qkv = jnp.dot(p, ipwT, preferred_element_type=jnp.float32) + ipb
        qs.append(qkv[:, 0:64])
        ks.append(qkv[:, 64:128])
        vs.append(qkv[:, 128:192])

    opwT = opwT_ref[...]
    opb = opb_ref[...]
    acc = None
    for i in range(4):
        parts = []
        for h in range(_HEADS):
            sl = slice(_DH * h, _DH * (h + 1))
            logits = [
                jnp.sum(qs[i][:, sl] * ks[j][:, sl], axis=1, keepdims=True)
                * 0.25
                for j in range(4)
            ]
            m = jnp.maximum(jnp.maximum(logits[0], logits[1]),
                            jnp.maximum(logits[2], logits[3]))
            es = [jnp.exp(l - m) for l in logits]
            den = es[0] + es[1] + es[2] + es[3]
            oh = (es[0] / den) * vs[0][:, sl]
            for j in range(1, 4):
                oh = oh + (es[j] / den) * vs[j][:, sl]
            parts.append(oh)
        o_i = jnp.concatenate(parts, axis=1)
        out_i = (jnp.dot(o_i, opwT, preferred_element_type=jnp.float32)
                 + opb)
        acc = out_i if acc is None else acc + out_i
    o_ref[...] = acc * 0.25


def _combine(cnt, s, s2, mx, mn, lin_w, lin_b, ipw, ipb, opw, opb):
    n = s.shape[0]
    lwT = jnp.transpose(lin_w, (0, 2, 1))
    lb = lin_b.reshape(4, 1, _NODE)
    ipwT = ipw.T
    ipb2 = ipb.reshape(1, -1)
    opwT = opw.T
    opb2 = opb.reshape(1, -1)
    return pl.pallas_call(
        _combiner_kernel,
        grid=(n // _NBLK,),
        in_specs=[
            pl.BlockSpec((_NBLK, 1), lambda i: (i, 0)),
            pl.BlockSpec((_NBLK, _NODE), lambda i: (i, 0)),
            pl.BlockSpec((_NBLK, _NODE), lambda i: (i, 0)),
            pl.BlockSpec((_NBLK, _NODE), lambda i: (i, 0)),
            pl.BlockSpec((_NBLK, _NODE), lambda i: (i, 0)),
            pl.BlockSpec((4, _NODE, _NODE), lambda i: (0, 0, 0)),
            pl.BlockSpec((4, 1, _NODE), lambda i: (0, 0, 0)),
            pl.BlockSpec((_NODE, 3 * _NODE), lambda i: (0, 0)),
            pl.BlockSpec((1, 3 * _NODE), lambda i: (0, 0)),
            pl.BlockSpec((_NODE, _NODE), lambda i: (0, 0)),
            pl.BlockSpec((1, _NODE), lambda i: (0, 0)),
        ],
        out_specs=pl.BlockSpec((_NBLK, _NODE), lambda i: (i, 0)),
        out_shape=jax.ShapeDtypeStruct((n, _NODE), jnp.float32),
    )(cnt, s, s2, mx, mn, lwT, lb, ipwT, ipb2, opwT, opb2)


def _colstats_kernel(z_ref, o_ref):
    @pl.when(pl.program_id(0) == 0)
    def _():
        o_ref[...] = jnp.zeros_like(o_ref)
    z = z_ref[...]
    o_ref[0:1, :] += jnp.sum(z, axis=0, keepdims=True)
    o_ref[1:2, :] += jnp.sum(z * z, axis=0, keepdims=True)


def _colstats(z, blk):
    n, d = z.shape
    return pl.pallas_call(
        _colstats_kernel,
        grid=(n // blk,),
        in_specs=[pl.BlockSpec((blk, d), lambda i: (i, 0))],
        out_specs=pl.BlockSpec((8, d), lambda i: (0, 0)),
        out_shape=jax.ShapeDtypeStruct((8, d), jnp.float32),
    )(z)


def _bn_scale_shift(stats, n, g, b):
    mean = stats[0] / n
    var = stats[1] / n - mean * mean
    sc = g / jnp.sqrt(var + 1e-5)
    sh = b - mean * sc
    return sc.reshape(1, -1), sh.reshape(1, -1)


def _zapply_kernel(z_ref, sc_ref, sh_ref, o_ref):
    zn = z_ref[...] * sc_ref[...] + sh_ref[...]
    o_ref[...] = _sigmoid(zn[:, :_NODE]) * _softplus(zn[:, _NODE:])


def _zapply(z, sc, sh):
    n = z.shape[0]
    return pl.pallas_call(
        _zapply_kernel,
        grid=(n // _EBLK,),
        in_specs=[
            pl.BlockSpec((_EBLK, 2 * _NODE), lambda i: (i, 0)),
            pl.BlockSpec((1, 2 * _NODE), lambda i: (0, 0)),
            pl.BlockSpec((1, 2 * _NODE), lambda i: (0, 0)),
        ],
        out_specs=pl.BlockSpec((_EBLK, _NODE), lambda i: (i, 0)),
        out_shape=jax.ShapeDtypeStruct((n, _NODE), jnp.float32),
    )(z, sc, sh)


def _oapply_kernel(u_ref, x_ref, sc_ref, sh_ref, o_ref):
    un = u_ref[...] * sc_ref[...] + sh_ref[...]
    o_ref[...] = _softplus(un + x_ref[...])


def _oapply(u, x, sc, sh):
    n = u.shape[0]
    return pl.pallas_call(
        _oapply_kernel,
        grid=(n // _NBLK,),
        in_specs=[
            pl.BlockSpec((_NBLK, _NODE), lambda i: (i, 0)),
            pl.BlockSpec((_NBLK, _NODE), lambda i: (i, 0)),
            pl.BlockSpec((1, _NODE), lambda i: (0, 0)),
            pl.BlockSpec((1, _NODE), lambda i: (0, 0)),
        ],
        out_specs=pl.BlockSpec((_NBLK, _NODE), lambda i: (i, 0)),
        out_shape=jax.ShapeDtypeStruct((n, _NODE), jnp.float32),
    )(u, x, sc, sh)


_NW = 32
_OWN = 1568
_SPAD = _NW * _OWN
_SCCH = 2000
_FL = 2048
_CH = 128


def _segstats_body(dst_hbm, src_hbm, table_hbm,
                   cnt_out, s_out, q_out, mx_out, mn_out, bind,
                   sdst, ssrc, cdst, bdst, gidx, rows,
                   acc_s, acc_q, acc_mx, acc_mn, sem):
    e = dst_hbm.shape[0]
    n_stream = e // _SCCH
    nvreg = _SCCH // 16
    w = lax.axis_index("s") * 2 + lax.axis_index("c")
    lo = pl.multiple_of(w * _OWN, 8)
    wbase = pl.multiple_of(w * (e + _FL), 2048)
    iota = lax.iota(jnp.int32, 16)

    def scan_vreg(k, carry):
        off, total = carry
        d = sdst[pl.ds(k * 16, 16)]
        s = ssrc[pl.ds(k * 16, 16)]
        m = (d >= lo) & (d < lo + _OWN)
        nmatch = jnp.sum(m.astype(jnp.int32))

        @pl.when(nmatch > 0)
        def _store():
            packed = jnp.where(m, s * 2048 + (d - lo),
                               jnp.int32(2147483647))
            pk_sorted, _unused = plsc.sort_key_val(packed, packed)
            cdst[pl.ds(off, 16)] = pk_sorted

        off = off + nmatch

        do_flush = off >= _FL

        @pl.when(do_flush)
        def _flush():
            pltpu.sync_copy(cdst.at[pl.ds(0, _FL)],
                            bind.at[pl.ds(pl.multiple_of(wbase + total, 2048), _FL)])
            cdst[pl.ds(0, 16)] = cdst[pl.ds(_FL, 16)]

        off = jnp.where(do_flush, off - _FL, off)
        total = jnp.where(do_flush, total + _FL, total)
        return off, total

    def scan_chunk(ci, carry):
        pltpu.sync_copy(dst_hbm.at[pl.ds(ci * _SCCH, _SCCH)], sdst)
        pltpu.sync_copy(src_hbm.at[pl.ds(ci * _SCCH, _SCCH)], ssrc)
        return lax.fori_loop(0, nvreg, scan_vreg, carry)

    off, total = lax.fori_loop(0, n_stream, scan_chunk, (0, 0))
    nm = total + off

    @pl.when(off > 0)
    def _drain():
        def poison(k2, _):
            posv = k2 * 16 + iota
            cur = cdst[pl.ds(k2 * 16, 16)]
            cdst[pl.ds(k2 * 16, 16)] = jnp.where(
                posv >= off, jnp.int32(2147483647), cur)
            return 0

        lax.fori_loop(0, _FL // 16, poison, 0)
        pltpu.sync_copy(cdst.at[pl.ds(0, _FL)],
                        bind.at[pl.ds(pl.multiple_of(wbase + total, 2048), _FL)])

    ncmax = (e + _CH - 1) // _CH

    for c in range(4):
        def init_row(j, _):
            acc_s[j] = jnp.zeros((16,), jnp.float32)
            acc_q[j] = jnp.zeros((16,), jnp.float32)
            acc_mx[j] = jnp.full((16,), -jnp.inf, jnp.float32)
            acc_mn[j] = jnp.full((16,), jnp.inf, jnp.float32)
            return 0

        lax.fori_loop(0, _OWN + 8, init_row, 0)

        def chunk_body(i, _, c=c):
            base = pl.multiple_of(i * _CH, _CH)

            @pl.when(base < nm)
            def _do():
                pltpu.sync_copy(
                    bind.at[pl.ds(pl.multiple_of(wbase + base, _CH), _CH)],
                    bdst.at[pl.ds(0, _CH)])
                rem = nm - base
                for j in range(_CH // 16):
                    sv = lax.shift_right_logical(bdst[pl.ds(j * 16, 16)], 12)
                    valid = (j * 16 + iota) < rem
                    gidx[0, pl.ds(j * 16, 16)] = jnp.where(valid, sv, 0)
                pltpu.async_copy(table_hbm.at[gidx.at[0]], rows, sem).wait()
                def rmw(i2, _):
                    pk = bdst[pl.ds(i2, 16)][0]
                    dl = jnp.minimum(jnp.bitwise_and(pk, 2047), _OWN)
                    half = jnp.bitwise_and(
                        lax.shift_right_logical(pk, 11), 1)
                    v = rows[i2, pl.ds(half * 64 + c * 16, 16)]
                    acc_s[dl] = acc_s[dl] + v
                    acc_q[dl] = acc_q[dl] + v * v
                    acc_mx[dl] = jnp.maximum(acc_mx[dl], v)
                    acc_mn[dl] = jnp.minimum(acc_mn[dl], v)
                    return 0

                lax.fori_loop(0, _CH, rmw, 0, unroll=4)

            return 0

        lax.fori_loop(0, ncmax, chunk_body, 0)
        obase = pl.multiple_of(c * _SPAD + lo, 8)
        pltpu.sync_copy(acc_s.at[pl.ds(0, _OWN)],
                        s_out.at[pl.ds(obase, _OWN), :])
        pltpu.sync_copy(acc_q.at[pl.ds(0, _OWN)],
                        q_out.at[pl.ds(obase, _OWN), :])
        pltpu.sync_copy(acc_mx.at[pl.ds(0, _OWN)],
                        mx_out.at[pl.ds(obase, _OWN), :])
        pltpu.sync_copy(acc_mn.at[pl.ds(0, _OWN)],
                        mn_out.at[pl.ds(obase, _OWN), :])

    def cnt_init(j, _):
        acc_s[j] = jnp.zeros((16,), jnp.float32)
        return 0

    lax.fori_loop(0, _OWN + 8, cnt_init, 0)

    def cnt_chunk(i, _):
        base = pl.multiple_of(i * _CH, _CH)

        @pl.when(base < nm)
        def _do():
            pltpu.sync_copy(
                bind.at[pl.ds(pl.multiple_of(wbase + base, _CH), _CH)],
                bdst.at[pl.ds(0, _CH)])
            def rmw(i2, _):
                dl = jnp.minimum(
                    jnp.bitwise_and(bdst[pl.ds(i2, 16)][0], 2047), _OWN)
                acc_s[dl] = acc_s[dl] + 1.0
                return 0

            lax.fori_loop(0, _CH, rmw, 0, unroll=4)

        return 0

    lax.fori_loop(0, ncmax, cnt_chunk, 0)
    pltpu.sync_copy(acc_s.at[pl.ds(0, _OWN)],
                    cnt_out.at[pl.ds(lo, _OWN), :])


def _segment_stats(dst, src, table, num_segments):
    e = dst.shape[0]
    table2 = table.reshape(-1, 128)
    ebin = e + _FL
    out_type = (
        pltpu.HBM((_SPAD, 16), jnp.float32),
        pltpu.HBM((4 * _SPAD, 16), jnp.float32),
        pltpu.HBM((4 * _SPAD, 16), jnp.float32),
        pltpu.HBM((4 * _SPAD, 16), jnp.float32),
        pltpu.HBM((4 * _SPAD, 16), jnp.float32),
        pltpu.HBM((_NW * ebin,), jnp.int32),
    )
    scratch = [
        pltpu.VMEM((_SCCH,), jnp.int32),
        pltpu.VMEM((_SCCH,), jnp.int32),
        pltpu.VMEM((_FL + 16,), jnp.int32),
        pltpu.VMEM((_CH + 16,), jnp.int32),
        pltpu.VMEM((1, 128), jnp.int32),
        pltpu.VMEM((_CH, 128), jnp.float32),
        pltpu.VMEM((_OWN + 8, 16), jnp.float32),
        pltpu.VMEM((_OWN + 8, 16), jnp.float32),
        pltpu.VMEM((_OWN + 8, 16), jnp.float32),
        pltpu.VMEM((_OWN + 8, 16), jnp.float32),
        pltpu.SemaphoreType.DMA,
    ]
    mesh = plsc.VectorSubcoreMesh(core_axis_name="c", subcore_axis_name="s")
    cnt16, s, q, mx, mn, _ = pl.kernel(
        _segstats_body, out_type=out_type, mesh=mesh,
        scratch_types=scratch,
        compiler_params=pltpu.CompilerParams(
            needs_layout_passes=False,
            use_tc_tiling_on_sc=False))(dst, src, table2)
    n = num_segments

    def asm(a):
        return a.reshape(4, _SPAD, 16).transpose(1, 0, 2).reshape(_SPAD, 64)[:n]

    return (cnt16[:n, :1], asm(s), asm(q), asm(mx), asm(mn))


def kernel(x, hyperedge_index, hyperedge_attrs,
           hedge_lin_w, hedge_lin_b, hedge_ipw, hedge_ipb, hedge_opw,
           hedge_opb, node_lin_w, node_lin_b, node_ipw, node_ipb, node_opw,
           node_opb, f2_w, f2_b, bn_f_g, bn_f_b, bn_c_g, bn_c_b, bn_o_g,
           bn_o_b):
    num_nodes = x.shape[0]
    num_hedges = hyperedge_attrs.shape[0]
    e = hyperedge_index.shape[1]
    hedge_idx = hyperedge_index[0]
    node_idx = hyperedge_index[1]

    cnt_h, s_h, s2_h, mx_h, mn_h = _segment_stats(
        hedge_idx, node_idx, x, num_hedges)
    hedge_out = _combine(cnt_h, s_h, s2_h, mx_h, mn_h,
                         hedge_lin_w, hedge_lin_b, hedge_ipw, hedge_ipb,
                         hedge_opw, hedge_opb)

    wi = f2_w[:, :_NODE].T
    wj_h = f2_w[:, _NODE:2 * _NODE].T
    wj_a = f2_w[:, 2 * _NODE:].T
    zero_b = jnp.zeros((1, 2 * _NODE), jnp.float32)
    xp = _mm(x, wi, zero_b)
    mp = _mm2(hedge_out, hyperedge_attrs, wj_h, wj_a,
              f2_b.reshape(1, -1))
    z = jnp.take(xp, node_idx, axis=0) + jnp.take(mp, hedge_idx, axis=0)

    zstats = _colstats(z, _EBLK)
    g2 = jnp.concatenate([bn_f_g, bn_c_g])
    b2 = jnp.concatenate([bn_f_b, bn_c_b])
    sc, sh = _bn_scale_shift(zstats, float(e), g2, b2)
    out_e = _zapply(z, sc, sh)

    eids = jnp.arange(e, dtype=jnp.int32)
    cnt_n, s_n, s2_n, mx_n, mn_n = _segment_stats(
        node_idx, eids, out_e, num_nodes)
    node_out = _combine(cnt_n, s_n, s2_n, mx_n, mn_n,
                        node_lin_w, node_lin_b, node_ipw, node_ipb,
                        node_opw, node_opb)

    ostats = _colstats(node_out, _NBLK)
    sco, sho = _bn_scale_shift(ostats, float(num_nodes), bn_o_g, bn_o_b)
    return _oapply(node_out, x, sco, sho)


# speedup vs baseline: 1.0681x; 1.0681x over previous
import functools

import jax
import jax.numpy as jnp
from jax import lax
from jax.experimental import pallas as pl
from jax.experimental.pallas import tpu as pltpu
from jax.experimental.pallas import tpu_sc as plsc

_NODE = 64
_HEADS = 4
_DH = 16
_NBLK = 400
_EBLK = 3200


def _softplus(x):
    return jnp.maximum(x, 0.0) + jnp.log1p(jnp.exp(-jnp.abs(x)))


def _sigmoid(x):
    return 1.0 / (1.0 + jnp.exp(-x))


def _mm_kernel(x_ref, w_ref, b_ref, o_ref):
    o_ref[...] = (
        jnp.dot(x_ref[...], w_ref[...], preferred_element_type=jnp.float32)
        + b_ref[...]
    )


def _mm(x, w, b, blk=_NBLK):
    n, k = x.shape
    m = w.shape[1]
    return pl.pallas_call(
        _mm_kernel,
        grid=(n // blk,),
        in_specs=[
            pl.BlockSpec((blk, k), lambda i: (i, 0)),
            pl.BlockSpec((k, m), lambda i: (0, 0)),
            pl.BlockSpec((1, m), lambda i: (0, 0)),
        ],
        out_specs=pl.BlockSpec((blk, m), lambda i: (i, 0)),
        out_shape=jax.ShapeDtypeStruct((n, m), jnp.float32),
    )(x, w, b)


def _mm2_kernel(a_ref, b_ref, wa_ref, wb_ref, bias_ref, o_ref):
    o_ref[...] = (
        jnp.dot(a_ref[...], wa_ref[...], preferred_element_type=jnp.float32)
        + jnp.dot(b_ref[...], wb_ref[...], preferred_element_type=jnp.float32)
        + bias_ref[...]
    )


def _mm2(a, b, wa, wb, bias, blk=_NBLK):
    n, ka = a.shape
    kb = b.shape[1]
    m = wa.shape[1]
    return pl.pallas_call(
        _mm2_kernel,
        grid=(n // blk,),
        in_specs=[
            pl.BlockSpec((blk, ka), lambda i: (i, 0)),
            pl.BlockSpec((blk, kb), lambda i: (i, 0)),
            pl.BlockSpec((ka, m), lambda i: (0, 0)),
            pl.BlockSpec((kb, m), lambda i: (0, 0)),
            pl.BlockSpec((1, m), lambda i: (0, 0)),
        ],
        out_specs=pl.BlockSpec((blk, m), lambda i: (i, 0)),
        out_shape=jax.ShapeDtypeStruct((n, m), jnp.float32),
    )(a, b, wa, wb, bias)


def _combiner_kernel(cnt_ref, s_ref, s2_ref, mx_ref, mn_ref,
                     lwT_ref, lb_ref, ipwT_ref, ipb_ref, opwT_ref, opb_ref,
                     o_ref):
    cnt = cnt_ref[...]
    cnt_c = jnp.maximum(cnt, 1.0)
    mean = s_ref[...] / cnt_c
    var = s2_ref[...] / cnt_c - mean * mean
    std = jnp.sqrt(jnp.clip(var, 1e-5, None))
    pos = cnt > 0.0
    mx = jnp.where(pos, mx_ref[...], 0.0)
    mn = jnp.where(pos, mn_ref[...], 0.0)
    aggs = (mean, std, mx, mn)

    ipwT = ipwT_ref[...]
    ipb = ipb_ref[...]
    qs, ks, vs = [], [], []
    for k in range(4):
        p = (jnp.dot(aggs[k], lwT_ref[k], preferred_element_type=jnp.float32)
             + lb_ref[k])
        qkv = jnp.dot(p, ipwT, preferred_element_type=jnp.float32) + ipb
        qs.append(qkv[:, 0:64])
        ks.append(qkv[:, 64:128])
        vs.append(qkv[:, 128:192])

    opwT = opwT_ref[...]
    opb = opb_ref[...]
    acc = None
    for i in range(4):
        parts = []
        for h in range(_HEADS):
            sl = slice(_DH * h, _DH * (h + 1))
            logits = [
                jnp.sum(qs[i][:, sl] * ks[j][:, sl], axis=1, keepdims=True)
                * 0.25
                for j in range(4)
            ]
            m = jnp.maximum(jnp.maximum(logits[0], logits[1]),
                            jnp.maximum(logits[2], logits[3]))
            es = [jnp.exp(l - m) for l in logits]
            den = es[0] + es[1] + es[2] + es[3]
            oh = (es[0] / den) * vs[0][:, sl]
            for j in range(1, 4):
                oh = oh + (es[j] / den) * vs[j][:, sl]
            parts.append(oh)
        o_i = jnp.concatenate(parts, axis=1)
        out_i = (jnp.dot(o_i, opwT, preferred_element_type=jnp.float32)
                 + opb)
        acc = out_i if acc is None else acc + out_i
    o_ref[...] = acc * 0.25


def _combine(cnt, s, s2, mx, mn, lin_w, lin_b, ipw, ipb, opw, opb):
    n = s.shape[0]
    lwT = jnp.transpose(lin_w, (0, 2, 1))
    lb = lin_b.reshape(4, 1, _NODE)
    ipwT = ipw.T
    ipb2 = ipb.reshape(1, -1)
    opwT = opw.T
    opb2 = opb.reshape(1, -1)
    return pl.pallas_call(
        _combiner_kernel,
        grid=(n // _NBLK,),
        in_specs=[
            pl.BlockSpec((_NBLK, 1), lambda i: (i, 0)),
            pl.BlockSpec((_NBLK, _NODE), lambda i: (i, 0)),
            pl.BlockSpec((_NBLK, _NODE), lambda i: (i, 0)),
            pl.BlockSpec((_NBLK, _NODE), lambda i: (i, 0)),
            pl.BlockSpec((_NBLK, _NODE), lambda i: (i, 0)),
            pl.BlockSpec((4, _NODE, _NODE), lambda i: (0, 0, 0)),
            pl.BlockSpec((4, 1, _NODE), lambda i: (0, 0, 0)),
            pl.BlockSpec((_NODE, 3 * _NODE), lambda i: (0, 0)),
            pl.BlockSpec((1, 3 * _NODE), lambda i: (0, 0)),
            pl.BlockSpec((_NODE, _NODE), lambda i: (0, 0)),
            pl.BlockSpec((1, _NODE), lambda i: (0, 0)),
        ],
        out_specs=pl.BlockSpec((_NBLK, _NODE), lambda i: (i, 0)),
        out_shape=jax.ShapeDtypeStruct((n, _NODE), jnp.float32),
    )(cnt, s, s2, mx, mn, lwT, lb, ipwT, ipb2, opwT, opb2)


def _colstats_kernel(z_ref, o_ref):
    @pl.when(pl.program_id(0) == 0)
    def _():
        o_ref[...] = jnp.zeros_like(o_ref)
    z = z_ref[...]
    o_ref[0:1, :] += jnp.sum(z, axis=0, keepdims=True)
    o_ref[1:2, :] += jnp.sum(z * z, axis=0, keepdims=True)


def _colstats(z, blk):
    n, d = z.shape
    return pl.pallas_call(
        _colstats_kernel,
        grid=(n // blk,),
        in_specs=[pl.BlockSpec((blk, d), lambda i: (i, 0))],
        out_specs=pl.BlockSpec((8, d), lambda i: (0, 0)),
        out_shape=jax.ShapeDtypeStruct((8, d), jnp.float32),
    )(z)


def _bn_scale_shift(stats, n, g, b):
    mean = stats[0] / n
    var = stats[1] / n - mean * mean
    sc = g / jnp.sqrt(var + 1e-5)
    sh = b - mean * sc
    return sc.reshape(1, -1), sh.reshape(1, -1)


def _zapply_kernel(z_ref, sc_ref, sh_ref, o_ref):
    zn = z_ref[...] * sc_ref[...] + sh_ref[...]
    o_ref[...] = _sigmoid(zn[:, :_NODE]) * _softplus(zn[:, _NODE:])


def _zapply(z, sc, sh):
    n = z.shape[0]
    return pl.pallas_call(
        _zapply_kernel,
        grid=(n // _EBLK,),
        in_specs=[
            pl.BlockSpec((_EBLK, 2 * _NODE), lambda i: (i, 0)),
            pl.BlockSpec((1, 2 * _NODE), lambda i: (0, 0)),
            pl.BlockSpec((1, 2 * _NODE), lambda i: (0, 0)),
        ],
        out_specs=pl.BlockSpec((_EBLK, _NODE), lambda i: (i, 0)),
        out_shape=jax.ShapeDtypeStruct((n, _NODE), jnp.float32),
    )(z, sc, sh)


def _oapply_kernel(u_ref, x_ref, sc_ref, sh_ref, o_ref):
    un = u_ref[...] * sc_ref[...] + sh_ref[...]
    o_ref[...] = _softplus(un + x_ref[...])


def _oapply(u, x, sc, sh):
    n = u.shape[0]
    return pl.pallas_call(
        _oapply_kernel,
        grid=(n // _NBLK,),
        in_specs=[
            pl.BlockSpec((_NBLK, _NODE), lambda i: (i, 0)),
            pl.BlockSpec((_NBLK, _NODE), lambda i: (i, 0)),
            pl.BlockSpec((1, _NODE), lambda i: (0, 0)),
            pl.BlockSpec((1, _NODE), lambda i: (0, 0)),
        ],
        out_specs=pl.BlockSpec((_NBLK, _NODE), lambda i: (i, 0)),
        out_shape=jax.ShapeDtypeStruct((n, _NODE), jnp.float32),
    )(u, x, sc, sh)


_NW = 32
_OWN = 1568
_SPAD = _NW * _OWN
_SCCH = 2000
_FL = 2048
_CH = 128


def _segstats_body(dst_hbm, src_hbm, table_hbm,
                   cnt_out, s_out, q_out, mx_out, mn_out, bind,
                   sdst, ssrc, cdst, bdst, gidx, rows,
                   acc_s, acc_q, acc_mx, acc_mn, sem):
    e = dst_hbm.shape[0]
    n_stream = e // _SCCH
    nvreg = _SCCH // 16
    w = lax.axis_index("s") * 2 + lax.axis_index("c")
    lo = pl.multiple_of(w * _OWN, 8)
    wbase = pl.multiple_of(w * (e + _FL), 2048)
    iota = lax.iota(jnp.int32, 16)

    def scan_vreg(k, carry):
        off, total = carry
        d = sdst[pl.ds(k * 16, 16)]
        s = ssrc[pl.ds(k * 16, 16)]
        m = (d >= lo) & (d < lo + _OWN)
        packed = jnp.where(m, s * 2048 + (d - lo), jnp.int32(2147483647))
        packed, _unused = plsc.sort_key_val(packed, packed)
        cdst[pl.ds(off, 16)] = packed
        off = off + jnp.sum(m.astype(jnp.int32))

        do_flush = off >= _FL

        @pl.when(do_flush)
        def _flush():
            pltpu.sync_copy(cdst.at[pl.ds(0, _FL)],
                            bind.at[pl.ds(pl.multiple_of(wbase + total, 2048), _FL)])
            cdst[pl.ds(0, 16)] = cdst[pl.ds(_FL, 16)]

        off = jnp.where(do_flush, off - _FL, off)
        total = jnp.where(do_flush, total + _FL, total)
        return off, total

    def scan_chunk(ci, carry):
        pltpu.sync_copy(dst_hbm.at[pl.ds(ci * _SCCH, _SCCH)], sdst)
        pltpu.sync_copy(src_hbm.at[pl.ds(ci * _SCCH, _SCCH)], ssrc)
        return lax.fori_loop(0, nvreg, scan_vreg, carry)

    off, total = lax.fori_loop(0, n_stream, scan_chunk, (0, 0))
    nm = total + off

    @pl.when(off > 0)
    def _drain():
        pltpu.sync_copy(cdst.at[pl.ds(0, _FL)],
                        bind.at[pl.ds(pl.multiple_of(wbase + total, 2048), _FL)])

    ncmax = (e + _CH - 1) // _CH

    for c in range(4):
        def init_row(j, _):
            acc_s[j] = jnp.zeros((16,), jnp.float32)
            acc_q[j] = jnp.zeros((16,), jnp.float32)
            acc_mx[j] = jnp.full((16,), -jnp.inf, jnp.float32)
            acc_mn[j] = jnp.full((16,), jnp.inf, jnp.float32)
            return 0

        lax.fori_loop(0, _OWN, init_row, 0)

        def chunk_body(i, _, c=c):
            base = pl.multiple_of(i * _CH, _CH)

            @pl.when(base < nm)
            def _do():
                pltpu.sync_copy(
                    bind.at[pl.ds(pl.multiple_of(wbase + base, _CH), _CH)],
                    bdst.at[pl.ds(0, _CH)])
                rem = nm - base
                for j in range(_CH // 16):
                    sv = lax.shift_right_logical(bdst[pl.ds(j * 16, 16)], 12)
                    valid = (j * 16 + iota) < rem
                    gidx[0, pl.ds(j * 16, 16)] = jnp.where(valid, sv, 0)
                pltpu.async_copy(table_hbm.at[gidx.at[0]], rows, sem).wait()
                ncnt = jnp.minimum(_CH, rem)

                def rmw(i2, _):
                    pk = bdst[pl.ds(i2, 16)][0]
                    dl = jnp.bitwise_and(pk, 2047)
                    half = jnp.bitwise_and(
                        lax.shift_right_logical(pk, 11), 1)
                    v = rows[i2, pl.ds(half * 64 + c * 16, 16)]
                    acc_s[dl] = acc_s[dl] + v
                    acc_q[dl] = acc_q[dl] + v * v
                    acc_mx[dl] = jnp.maximum(acc_mx[dl], v)
                    acc_mn[dl] = jnp.minimum(acc_mn[dl], v)
                    return 0

                lax.fori_loop(0, ncnt, rmw, 0)

            return 0

        lax.fori_loop(0, ncmax, chunk_body, 0)
        obase = pl.multiple_of(c * _SPAD + lo, 8)
        pltpu.sync_copy(acc_s, s_out.at[pl.ds(obase, _OWN), :])
        pltpu.sync_copy(acc_q, q_out.at[pl.ds(obase, _OWN), :])
        pltpu.sync_copy(acc_mx, mx_out.at[pl.ds(obase, _OWN), :])
        pltpu.sync_copy(acc_mn, mn_out.at[pl.ds(obase, _OWN), :])

    def cnt_init(j, _):
        acc_s[j] = jnp.zeros((16,), jnp.float32)
        return 0

    lax.fori_loop(0, _OWN, cnt_init, 0)

    def cnt_chunk(i, _):
        base = pl.multiple_of(i * _CH, _CH)

        @pl.when(base < nm)
        def _do():
            pltpu.sync_copy(
                bind.at[pl.ds(pl.multiple_of(wbase + base, _CH), _CH)],
                bdst.at[pl.ds(0, _CH)])
            ncnt = jnp.minimum(_CH, nm - base)

            def rmw(i2, _):
                dl = jnp.bitwise_and(bdst[pl.ds(i2, 16)][0], 2047)
                acc_s[dl] = acc_s[dl] + 1.0
                return 0

            lax.fori_loop(0, ncnt, rmw, 0)

        return 0

    lax.fori_loop(0, ncmax, cnt_chunk, 0)
    pltpu.sync_copy(acc_s, cnt_out.at[pl.ds(lo, _OWN), :])


def _segment_stats(dst, src, table, num_segments):
    e = dst.shape[0]
    table2 = table.reshape(-1, 128)
    ebin = e + _FL
    out_type = (
        pltpu.HBM((_SPAD, 16), jnp.float32),
        pltpu.HBM((4 * _SPAD, 16), jnp.float32),
        pltpu.HBM((4 * _SPAD, 16), jnp.float32),
        pltpu.HBM((4 * _SPAD, 16), jnp.float32),
        pltpu.HBM((4 * _SPAD, 16), jnp.float32),
        pltpu.HBM((_NW * ebin,), jnp.int32),
    )
    scratch = [
        pltpu.VMEM((_SCCH,), jnp.int32),
        pltpu.VMEM((_SCCH,), jnp.int32),
        pltpu.VMEM((_FL + 16,), jnp.int32),
        pltpu.VMEM((_CH + 16,), jnp.int32),
        pltpu.VMEM((1, 128), jnp.int32),
        pltpu.VMEM((_CH, 128), jnp.float32),
        pltpu.VMEM((_OWN, 16), jnp.float32),
        pltpu.VMEM((_OWN, 16), jnp.float32),
        pltpu.VMEM((_OWN, 16), jnp.float32),
        pltpu.VMEM((_OWN, 16), jnp.float32),
        pltpu.SemaphoreType.DMA,
    ]
    mesh = plsc.VectorSubcoreMesh(core_axis_name="c", subcore_axis_name="s")
    cnt16, s, q, mx, mn, _ = pl.kernel(
        _segstats_body, out_type=out_type, mesh=mesh,
        scratch_types=scratch,
        compiler_params=pltpu.CompilerParams(
            needs_layout_passes=False,
            use_tc_tiling_on_sc=False))(dst, src, table2)
    n = num_segments

    def asm(a):
        return a.reshape(4, _SPAD, 16).transpose(1, 0, 2).reshape(_SPAD, 64)[:n]

    return (cnt16[:n, :1], asm(s), asm(q), asm(mx), asm(mn))


def kernel(x, hyperedge_index, hyperedge_attrs,
           hedge_lin_w, hedge_lin_b, hedge_ipw, hedge_ipb, hedge_opw,
           hedge_opb, node_lin_w, node_lin_b, node_ipw, node_ipb, node_opw,
           node_opb, f2_w, f2_b, bn_f_g, bn_f_b, bn_c_g, bn_c_b, bn_o_g,
           bn_o_b):
    num_nodes = x.shape[0]
    num_hedges = hyperedge_attrs.shape[0]
    e = hyperedge_index.shape[1]
    hedge_idx = hyperedge_index[0]
    node_idx = hyperedge_index[1]

    cnt_h, s_h, s2_h, mx_h, mn_h = _segment_stats(
        hedge_idx, node_idx, x, num_hedges)
    hedge_out = _combine(cnt_h, s_h, s2_h, mx_h, mn_h,
                         hedge_lin_w, hedge_lin_b, hedge_ipw, hedge_ipb,
                         hedge_opw, hedge_opb)

    wi = f2_w[:, :_NODE].T
    wj_h = f2_w[:, _NODE:2 * _NODE].T
    wj_a = f2_w[:, 2 * _NODE:].T
    zero_b = jnp.zeros((1, 2 * _NODE), jnp.float32)
    xp = _mm(x, wi, zero_b)
    mp = _mm2(hedge_out, hyperedge_attrs, wj_h, wj_a,
              f2_b.reshape(1, -1))
    z = jnp.take(xp, node_idx, axis=0) + jnp.take(mp, hedge_idx, axis=0)

    zstats = _colstats(z, _EBLK)
    g2 = jnp.concatenate([bn_f_g, bn_c_g])
    b2 = jnp.concatenate([bn_f_b, bn_c_b])
    sc, sh = _bn_scale_shift(zstats, float(e), g2, b2)
    out_e = _zapply(z, sc, sh)

    eids = jnp.arange(e, dtype=jnp.int32)
    cnt_n, s_n, s2_n, mx_n, mn_n = _segment_stats(
        node_idx, eids, out_e, num_nodes)
    node_out = _combine(cnt_n, s_n, s2_n, mx_n, mn_n,
                        node_lin_w, node_lin_b, node_ipw, node_ipb,
                        node_opw, node_opb)

    ostats = _colstats(node_out, _NBLK)
    sco, sho = _bn_scale_shift(ostats, float(num_nodes), bn_o_g, bn_o_b)
    return _oapply(node_out, x, sco, sho)
